# Initial kernel scaffold; baseline (speedup 1.0000x reference)
import jax
import jax.numpy as jnp
from jax.experimental import pallas as pl


def kernel(con, feat):
    raise NotImplementedError("write your pallas kernel here")


# speedup vs baseline: 8274.1507x; 8274.1507x over previous
import jax
import jax.numpy as jnp
from jax.experimental import pallas as pl
from jax.experimental.pallas import tpu as pltpu

_L = 1024
_NEG = -1e30
_BIGK = 3e7


def _pairs_from_rows(seq, axis):
    iota = jax.lax.broadcasted_iota(jnp.int32, seq.shape, axis)
    m = jnp.max(seq, axis=axis, keepdims=True)
    cls = jnp.min(jnp.where(seq == m, iota, 4), axis=axis, keepdims=True)
    return jnp.where(
        cls == 0, 2.0, jnp.where(cls == 1, 3.0, jnp.where(cls == 2, 5.0, 7.0))
    ).astype(jnp.float32)


def _pp_kernel(con_ref, conT_ref, seq_ref, seqT_ref, out_ref, w_ref, k_ref):
    con = con_ref[...]
    conT = conT_ref[...]

    pR = _pairs_from_rows(seq_ref[...], 0)
    pC = _pairs_from_rows(seqT_ref[...], 1)
    prod = pC * pR
    allowed = (prod == 14.0) | (prod == 15.0) | (prod == 35.0)

    ii = jax.lax.broadcasted_iota(jnp.int32, (_L, _L), 0)
    jj = jax.lax.broadcasted_iota(jnp.int32, (_L, _L), 1)
    upper = jj - ii >= 5
    band = upper | (ii - jj >= 5)
    w = jnp.where(band & allowed, jnp.where(upper, con, conT), 0.0)
    w_ref[...] = w
    ekey = (
        jnp.minimum(ii, jj).astype(jnp.float32) * 1024.0
        + jnp.maximum(ii, jj).astype(jnp.float32)
    )
    k_ref[...] = ekey
    out_ref[...] = jnp.zeros((_L, _L), jnp.float32)

    def cond(carry):
        gmax, _, _ = carry
        return gmax > 0.0

    def body(carry):
        _, free_r, free_c = carry
        w = w_ref[...]
        ekey = k_ref[...]
        live = (free_r > 0.5) & (free_c > 0.5)
        wm = jnp.where(live, w, _NEG)
        bv_c = jnp.max(wm, axis=1, keepdims=True)
        bv_r = jnp.max(wm, axis=0, keepdims=True)
        at_c = wm == bv_c
        at_r = wm == bv_r
        km_c = jnp.where(at_c, ekey, _BIGK)
        km_r = jnp.where(at_r, ekey, _BIGK)
        bk_c = jnp.min(km_c, axis=1, keepdims=True)
        bk_r = jnp.min(km_r, axis=0, keepdims=True)
        rec = (
            (wm > 0.0)
            & at_c & (ekey == bk_c)
            & at_r & (ekey == bk_r)
        )
        out_ref[...] += jnp.where(rec, w, 0.0)
        recf = rec.astype(jnp.float32)
        free_c2 = free_c * (1.0 - jnp.max(recf, axis=1, keepdims=True))
        free_r2 = free_r * (1.0 - jnp.max(recf, axis=0, keepdims=True))
        live2 = (free_r2 > 0.5) & (free_c2 > 0.5)
        gmax2 = jnp.max(jnp.where(live2, w, _NEG))
        return gmax2, free_r2, free_c2

    carry0 = (
        jnp.max(w),
        jnp.ones((1, _L), jnp.float32),
        jnp.ones((_L, 1), jnp.float32),
    )
    jax.lax.while_loop(cond, body, carry0)


def kernel(con, feat):
    con2d = con.reshape(_L, _L)
    conT = jnp.swapaxes(con2d, 0, 1)
    seq = feat[0, :4, :, 0]
    seqT = jnp.swapaxes(seq, 0, 1)
    out = pl.pallas_call(
        _pp_kernel,
        out_shape=jax.ShapeDtypeStruct((_L, _L), jnp.float32),
        scratch_shapes=[
            pltpu.VMEM((_L, _L), jnp.float32),
            pltpu.VMEM((_L, _L), jnp.float32),
        ],
    )(con2d, conT, seq, seqT)
    return out.reshape(con.shape)
